# baseline (speedup 1.0000x reference)
import functools

import jax
import jax.numpy as jnp
import numpy as np
from jax.experimental import pallas as pl
from jax.experimental.pallas import tpu as pltpu
from jax.experimental.pallas import tpu_sc as plsc

B = 1024
DIM_X = 256
DIM_Y = 64
NUM_EIG = 32
EIG_OUT = 64
EIG_IN = 16
EPS = 1e-10

RB = 64
NBLK = B // RB
NC = NUM_EIG * EIG_IN
LANES = RB


def _ce(kv, kc, riota, dist, k_sz):
    n, L = kv.shape
    g = n // (2 * dist)
    kv4 = kv.reshape(g, 2, dist, L)
    kc4 = kc.reshape(g, 2, dist, L)
    av, bv = kv4[:, 0], kv4[:, 1]
    ac, bc = kc4[:, 0], kc4[:, 1]
    a_first = (av > bv) | ((av == bv) & (ac < bc))
    if k_sz is None:
        keep = a_first
    else:
        r4 = riota.reshape(g, 2, dist, L)[:, 0]
        desc = (jax.lax.shift_right_logical(r4, int(np.log2(k_sz)))
                & jnp.int32(1)) == 0
        keep = desc == a_first
    nav = jnp.where(keep, av, bv)
    nbv = jnp.where(keep, bv, av)
    nac = jnp.where(keep, ac, bc)
    nbc = jnp.where(keep, bc, ac)
    kv_n = jnp.stack([nav, nbv], axis=1).reshape(n, L)
    kc_n = jnp.stack([nac, nbc], axis=1).reshape(n, L)
    return kv_n, kc_n


def _top64_sorted(kv, kc):
    n = NC
    riota = jax.lax.broadcasted_iota(jnp.int32, (n, LANES), 0)
    for k_sz in (2, 4, 8, 16, 32, 64):
        d = k_sz // 2
        while d >= 1:
            kv, kc = _ce(kv, kc, riota, d, k_sz)
            d //= 2
    while n > 64:
        kv, kc = _ce(kv, kc, riota, 64, None)
        kv = kv.reshape(n // 128, 2, 64, LANES)[:, 0].reshape(n // 2, LANES)
        kc = kc.reshape(n // 128, 2, 64, LANES)[:, 0].reshape(n // 2, LANES)
        n //= 2
        riota = jax.lax.broadcasted_iota(jnp.int32, (n, LANES), 0)
        for d in (32, 16, 8, 4, 2, 1):
            kv, kc = _ce(kv, kc, riota, d, 64)
    return kv, kc


def _tc_body(lhsT_ref, w1_ref, ev2_ref, in_w_ref, wb_ref,
             y_ref, vals_ref, idx_ref):
    b0 = pl.program_id(0)
    LT = lhsT_ref[...]
    p1 = jax.lax.dot_general(w1_ref[...], LT, (((1,), (0,)), ((), ())),
                             preferred_element_type=jnp.float32)
    acc = jnp.square(p1[0:NUM_EIG])
    for k in range(1, DIM_Y):
        acc = acc + jnp.square(p1[k * NUM_EIG:(k + 1) * NUM_EIG])
    y_norm = jnp.sqrt(acc)
    w_col = wb_ref[:, 0:1]
    t = w_col * jnp.square(y_norm)
    s = t[0:1]
    for l in range(1, NUM_EIG):
        s = s + t[l:l + 1]
    s = jnp.maximum(s, EPS)
    ow = in_w_ref[...] * (t / s)
    x = jnp.concatenate([ow[:, i * RB:(i + 1) * RB] for i in range(EIG_IN)],
                        axis=0)
    kv = jax.lax.bitcast_convert_type(x, jnp.int32)
    kv = kv ^ jax.lax.shift_right_arithmetic(kv, 31) & jnp.int32(0x7FFFFFFF)
    r2 = jax.lax.broadcasted_iota(jnp.int32, (NC, RB), 0)
    kc = jax.lax.shift_left(r2 & jnp.int32(NUM_EIG - 1), 4) | \
        jax.lax.shift_right_logical(r2, 5)
    kv, kc = _top64_sorted(kv, kc)
    vb = kv ^ jax.lax.shift_right_arithmetic(kv, 31) & jnp.int32(0x7FFFFFFF)
    vals = jax.lax.bitcast_convert_type(vb, jnp.float32)
    vsum = jnp.sum(vals, axis=0, keepdims=True)
    vals_ref[...] = jnp.transpose(vals / vsum)
    i_part = kc & jnp.int32(EIG_IN - 1)
    l_part = jax.lax.shift_right_logical(kc, 4)
    r_iota = jax.lax.broadcasted_iota(jnp.int32, (EIG_OUT, RB), 1)
    gidx = (b0.astype(jnp.int32) * (RB * EIG_IN * NUM_EIG)
            + i_part * (RB * NUM_EIG) + r_iota * NUM_EIG + l_part)
    idx_ref[...] = jnp.transpose(gidx)
    p2 = jax.lax.dot_general(LT, ev2_ref[...], (((0,), (0,)), ((), ())),
                             preferred_element_type=jnp.float32)
    normc = jnp.maximum(y_norm, EPS)
    norm_nat = jnp.transpose(normc)
    for l in range(NUM_EIG):
        sl = slice(l * DIM_Y, (l + 1) * DIM_Y)
        y_ref[:, sl] = p2[:, sl] / jnp.broadcast_to(
            norm_nat[:, l:l + 1], (RB * EIG_IN, DIM_Y))


@functools.partial(jax.jit, static_argnums=())
def _tc_stage(lhsT_bf, w1_bf, ev2_bf, in_w_perm, w_big):
    return pl.pallas_call(
        _tc_body,
        grid=(NBLK,),
        in_specs=[
            pl.BlockSpec((DIM_X, RB * EIG_IN), lambda b: (0, b)),
            pl.BlockSpec((DIM_Y * NUM_EIG, DIM_X), lambda b: (0, 0)),
            pl.BlockSpec((DIM_X, NUM_EIG * DIM_Y), lambda b: (0, 0)),
            pl.BlockSpec((1, RB * EIG_IN), lambda b: (0, b)),
            pl.BlockSpec((NUM_EIG, 128), lambda b: (0, 0)),
        ],
        out_specs=[
            pl.BlockSpec((RB * EIG_IN, NUM_EIG * DIM_Y), lambda b: (b, 0)),
            pl.BlockSpec((RB, EIG_OUT), lambda b: (b, 0)),
            pl.BlockSpec((RB, EIG_OUT), lambda b: (b, 0)),
        ],
        out_shape=[
            jax.ShapeDtypeStruct((B * EIG_IN, NUM_EIG * DIM_Y), jnp.float32),
            jax.ShapeDtypeStruct((B, EIG_OUT), jnp.float32),
            jax.ShapeDtypeStruct((B, EIG_OUT), jnp.int32),
        ],
        compiler_params=pltpu.CompilerParams(
            dimension_semantics=("parallel",)),
    )(lhsT_bf, w1_bf, ev2_bf, in_w_perm, w_big)


def _sc_gather(table, indices):
    n_idx = indices.shape[0] * indices.shape[1]
    idx2 = indices.reshape(1, n_idx)
    gw = 128

    @functools.partial(
        pl.kernel,
        out_type=jax.ShapeDtypeStruct((n_idx, 2 * DIM_Y), jnp.float32),
        mesh=plsc.VectorSubcoreMesh(core_axis_name="core",
                                    subcore_axis_name="subcore"))
    def kernel(tab_hbm, i_hbm, o_hbm):
        def body(i_vmem, o_vmem):
            pltpu.sync_copy(tab_hbm.at[i_vmem.at[0]], o_vmem)

        pltpu.emit_pipeline(
            body,
            grid=(n_idx // gw,),
            in_specs=[pl.BlockSpec((1, gw), index_map=lambda i: (0, i))],
            out_specs=[pl.BlockSpec((gw, 2 * DIM_Y),
                                    index_map=lambda i: (i, 0))],
            core_axis_name=("core", "subcore"),
            dimension_semantics=(pltpu.PARALLEL,),
        )(i_hbm, o_hbm)

    return kernel(table, idx2)


SB = 32


def _asm_body(yg_ref, gidx_ref, vals_ref, out_ref):
    par_t = jnp.transpose(gidx_ref[...] & jnp.int32(1))
    for s in range(SB):
        y2 = yg_ref[s * EIG_OUT:(s + 1) * EIG_OUT, :]
        m = jnp.broadcast_to(par_t[:, s:s + 1] == 1, (EIG_OUT, DIM_Y))
        sel = jnp.where(m, y2[:, DIM_Y:], y2[:, :DIM_Y])
        out_ref[s * (DIM_Y + 1) + 1:(s + 1) * (DIM_Y + 1), :] = jnp.transpose(sel)
        out_ref[s * (DIM_Y + 1):s * (DIM_Y + 1) + 1, :] = vals_ref[s:s + 1, :]


def _asm_stage(yg128, gidx, vals):
    nblk = B // SB
    return pl.pallas_call(
        _asm_body,
        grid=(nblk,),
        in_specs=[
            pl.BlockSpec((SB * EIG_OUT, 2 * DIM_Y), lambda b: (b, 0)),
            pl.BlockSpec((SB, EIG_OUT), lambda b: (b, 0)),
            pl.BlockSpec((SB, EIG_OUT), lambda b: (b, 0)),
        ],
        out_specs=pl.BlockSpec((SB * (DIM_Y + 1), DIM_Y), lambda b: (b, 0)),
        out_shape=jax.ShapeDtypeStruct((B * (DIM_Y + 1), DIM_Y), jnp.float32),
        compiler_params=pltpu.CompilerParams(
            dimension_semantics=("parallel",)),
    )(yg128, gidx, vals)


def kernel(inputs, eig_vec, eig_val):
    norms = jnp.linalg.norm(eig_vec, axis=0, keepdims=True)
    ev = eig_vec / norms
    w = jax.nn.relu(eig_val)
    w = w / jnp.sum(w)
    ev3 = ev.reshape(DIM_X, DIM_Y, NUM_EIG)
    w1_bf = ev3.transpose(1, 2, 0).reshape(DIM_Y * NUM_EIG, DIM_X).astype(jnp.bfloat16)
    ev2_bf = ev3.transpose(0, 2, 1).reshape(DIM_X, NUM_EIG * DIM_Y).astype(jnp.bfloat16)
    in_v = inputs[:, 1:, :]
    lhsT_bf = (in_v.reshape(NBLK, RB, DIM_X, EIG_IN)
               .transpose(2, 0, 3, 1).reshape(DIM_X, B * EIG_IN)
               .astype(jnp.bfloat16))
    in_w_perm = (inputs[:, 0, :].reshape(NBLK, RB, EIG_IN)
                 .transpose(0, 2, 1).reshape(1, B * EIG_IN))
    w_big = jnp.tile(w[:, None], (1, 128))

    y_tab, vals, idx = _tc_stage(lhsT_bf, w1_bf, ev2_bf, in_w_perm, w_big)
    y_tab = y_tab.reshape(B * EIG_IN * NUM_EIG // 2, 2 * DIM_Y)
    rows = jax.lax.shift_right_logical(idx, 1)
    yg128 = _sc_gather(y_tab, rows)
    out = _asm_stage(yg128, idx, vals)
    return out.reshape(B, DIM_Y + 1, DIM_Y)


# speedup vs baseline: 1.1838x; 1.1838x over previous
import functools

import jax
import jax.numpy as jnp
import numpy as np
from jax.experimental import pallas as pl
from jax.experimental.pallas import tpu as pltpu
from jax.experimental.pallas import tpu_sc as plsc

B = 1024
DIM_X = 256
DIM_Y = 64
NUM_EIG = 32
EIG_OUT = 64
EIG_IN = 16
EPS = 1e-10

RB = 128
NBLK = B // RB
NC = NUM_EIG * EIG_IN
LANES = RB


def _ce(kv, kc, riota, dist, k_sz):
    n, L = kv.shape
    g = n // (2 * dist)
    kv4 = kv.reshape(g, 2, dist, L)
    kc4 = kc.reshape(g, 2, dist, L)
    av, bv = kv4[:, 0], kv4[:, 1]
    ac, bc = kc4[:, 0], kc4[:, 1]
    a_first = (av > bv) | ((av == bv) & (ac < bc))
    if k_sz is None:
        keep = a_first
    else:
        r4 = riota.reshape(g, 2, dist, L)[:, 0]
        desc = (jax.lax.shift_right_logical(r4, int(np.log2(k_sz)))
                & jnp.int32(1)) == 0
        keep = desc == a_first
    nav = jnp.where(keep, av, bv)
    nbv = jnp.where(keep, bv, av)
    nac = jnp.where(keep, ac, bc)
    nbc = jnp.where(keep, bc, ac)
    kv_n = jnp.stack([nav, nbv], axis=1).reshape(n, L)
    kc_n = jnp.stack([nac, nbc], axis=1).reshape(n, L)
    return kv_n, kc_n


def _top64_sorted(kv, kc):
    n = NC
    riota = jax.lax.broadcasted_iota(jnp.int32, (n, LANES), 0)
    for k_sz in (2, 4, 8, 16, 32, 64):
        d = k_sz // 2
        while d >= 1:
            kv, kc = _ce(kv, kc, riota, d, k_sz)
            d //= 2
    while n > 64:
        kv, kc = _ce(kv, kc, riota, 64, None)
        kv = kv.reshape(n // 128, 2, 64, LANES)[:, 0].reshape(n // 2, LANES)
        kc = kc.reshape(n // 128, 2, 64, LANES)[:, 0].reshape(n // 2, LANES)
        n //= 2
        riota = jax.lax.broadcasted_iota(jnp.int32, (n, LANES), 0)
        for d in (32, 16, 8, 4, 2, 1):
            kv, kc = _ce(kv, kc, riota, d, 64)
    return kv, kc


def _tc_body(lhsT_ref, w1_ref, ev2_ref, in_w_ref, wb_ref,
             y_ref, vals_ref, idx_ref):
    b0 = pl.program_id(0)
    LT = lhsT_ref[...]
    p1 = jax.lax.dot_general(w1_ref[...], LT, (((1,), (0,)), ((), ())),
                             preferred_element_type=jnp.float32)
    acc = jnp.square(p1[0:NUM_EIG])
    for k in range(1, DIM_Y):
        acc = acc + jnp.square(p1[k * NUM_EIG:(k + 1) * NUM_EIG])
    y_norm = jnp.sqrt(acc)
    w_col = wb_ref[:, 0:1]
    t = w_col * jnp.square(y_norm)
    s = t[0:1]
    for l in range(1, NUM_EIG):
        s = s + t[l:l + 1]
    s = jnp.maximum(s, EPS)
    ow = in_w_ref[...] * (t / s)
    x = jnp.concatenate([ow[:, i * RB:(i + 1) * RB] for i in range(EIG_IN)],
                        axis=0)
    kv = jax.lax.bitcast_convert_type(x, jnp.int32)
    kv = kv ^ jax.lax.shift_right_arithmetic(kv, 31) & jnp.int32(0x7FFFFFFF)
    r2 = jax.lax.broadcasted_iota(jnp.int32, (NC, RB), 0)
    kc = jax.lax.shift_left(r2 & jnp.int32(NUM_EIG - 1), 4) | \
        jax.lax.shift_right_logical(r2, 5)
    kv, kc = _top64_sorted(kv, kc)
    vb = kv ^ jax.lax.shift_right_arithmetic(kv, 31) & jnp.int32(0x7FFFFFFF)
    vals = jax.lax.bitcast_convert_type(vb, jnp.float32)
    vsum = jnp.sum(vals, axis=0, keepdims=True)
    vals_ref[...] = jnp.transpose(vals / vsum)
    i_part = kc & jnp.int32(EIG_IN - 1)
    l_part = jax.lax.shift_right_logical(kc, 4)
    r_iota = jax.lax.broadcasted_iota(jnp.int32, (EIG_OUT, RB), 1)
    gidx = (b0.astype(jnp.int32) * (RB * EIG_IN * NUM_EIG)
            + i_part * (RB * NUM_EIG) + r_iota * NUM_EIG + l_part)
    idx_ref[...] = jnp.transpose(gidx)
    p2 = jax.lax.dot_general(LT, ev2_ref[...], (((0,), (0,)), ((), ())),
                             preferred_element_type=jnp.float32)
    normc = jnp.maximum(y_norm, EPS)
    norm_nat = jnp.transpose(normc)
    for l in range(NUM_EIG):
        sl = slice(l * DIM_Y, (l + 1) * DIM_Y)
        y_ref[:, sl] = p2[:, sl] / jnp.broadcast_to(
            norm_nat[:, l:l + 1], (RB * EIG_IN, DIM_Y))


@functools.partial(jax.jit, static_argnums=())
def _tc_stage(lhsT_bf, w1_bf, ev2_bf, in_w_perm, w_big):
    return pl.pallas_call(
        _tc_body,
        grid=(NBLK,),
        in_specs=[
            pl.BlockSpec((DIM_X, RB * EIG_IN), lambda b: (0, b)),
            pl.BlockSpec((DIM_Y * NUM_EIG, DIM_X), lambda b: (0, 0)),
            pl.BlockSpec((DIM_X, NUM_EIG * DIM_Y), lambda b: (0, 0)),
            pl.BlockSpec((1, RB * EIG_IN), lambda b: (0, b)),
            pl.BlockSpec((NUM_EIG, 128), lambda b: (0, 0)),
        ],
        out_specs=[
            pl.BlockSpec((RB * EIG_IN, NUM_EIG * DIM_Y), lambda b: (b, 0)),
            pl.BlockSpec((RB, EIG_OUT), lambda b: (b, 0)),
            pl.BlockSpec((RB, EIG_OUT), lambda b: (b, 0)),
        ],
        out_shape=[
            jax.ShapeDtypeStruct((B * EIG_IN, NUM_EIG * DIM_Y), jnp.float32),
            jax.ShapeDtypeStruct((B, EIG_OUT), jnp.float32),
            jax.ShapeDtypeStruct((B, EIG_OUT), jnp.int32),
        ],
        compiler_params=pltpu.CompilerParams(
            dimension_semantics=("parallel",)),
    )(lhsT_bf, w1_bf, ev2_bf, in_w_perm, w_big)


def _sc_gather(table, indices):
    n_idx = indices.shape[0] * indices.shape[1]
    idx2 = indices.reshape(1, n_idx)
    gw = 128

    @functools.partial(
        pl.kernel,
        out_type=jax.ShapeDtypeStruct((n_idx, 2 * DIM_Y), jnp.float32),
        mesh=plsc.VectorSubcoreMesh(core_axis_name="core",
                                    subcore_axis_name="subcore"))
    def kernel(tab_hbm, i_hbm, o_hbm):
        def body(i_vmem, o_vmem):
            pltpu.sync_copy(tab_hbm.at[i_vmem.at[0]], o_vmem)

        pltpu.emit_pipeline(
            body,
            grid=(n_idx // gw,),
            in_specs=[pl.BlockSpec((1, gw), index_map=lambda i: (0, i))],
            out_specs=[pl.BlockSpec((gw, 2 * DIM_Y),
                                    index_map=lambda i: (i, 0))],
            core_axis_name=("core", "subcore"),
            dimension_semantics=(pltpu.PARALLEL,),
        )(i_hbm, o_hbm)

    return kernel(table, idx2)


SB = 32


def _asm_body(yg_ref, gidx_ref, vals_ref, out_ref):
    par_t = jnp.transpose(gidx_ref[...] & jnp.int32(1))
    for s in range(SB):
        y2 = yg_ref[s * EIG_OUT:(s + 1) * EIG_OUT, :]
        m = jnp.broadcast_to(par_t[:, s:s + 1] == 1, (EIG_OUT, DIM_Y))
        sel = jnp.where(m, y2[:, DIM_Y:], y2[:, :DIM_Y])
        out_ref[s * (DIM_Y + 1) + 1:(s + 1) * (DIM_Y + 1), :] = jnp.transpose(sel)
        out_ref[s * (DIM_Y + 1):s * (DIM_Y + 1) + 1, :] = vals_ref[s:s + 1, :]


def _asm_stage(yg128, gidx, vals):
    nblk = B // SB
    return pl.pallas_call(
        _asm_body,
        grid=(nblk,),
        in_specs=[
            pl.BlockSpec((SB * EIG_OUT, 2 * DIM_Y), lambda b: (b, 0)),
            pl.BlockSpec((SB, EIG_OUT), lambda b: (b, 0)),
            pl.BlockSpec((SB, EIG_OUT), lambda b: (b, 0)),
        ],
        out_specs=pl.BlockSpec((SB * (DIM_Y + 1), DIM_Y), lambda b: (b, 0)),
        out_shape=jax.ShapeDtypeStruct((B * (DIM_Y + 1), DIM_Y), jnp.float32),
        compiler_params=pltpu.CompilerParams(
            dimension_semantics=("parallel",)),
    )(yg128, gidx, vals)


def kernel(inputs, eig_vec, eig_val):
    norms = jnp.linalg.norm(eig_vec, axis=0, keepdims=True)
    ev = eig_vec / norms
    w = jax.nn.relu(eig_val)
    w = w / jnp.sum(w)
    ev3 = ev.reshape(DIM_X, DIM_Y, NUM_EIG)
    w1_bf = ev3.transpose(1, 2, 0).reshape(DIM_Y * NUM_EIG, DIM_X).astype(jnp.bfloat16)
    ev2_bf = ev3.transpose(0, 2, 1).reshape(DIM_X, NUM_EIG * DIM_Y).astype(jnp.bfloat16)
    in_v = inputs[:, 1:, :]
    lhsT_bf = (in_v.reshape(NBLK, RB, DIM_X, EIG_IN)
               .transpose(2, 0, 3, 1).reshape(DIM_X, B * EIG_IN)
               .astype(jnp.bfloat16))
    in_w_perm = (inputs[:, 0, :].reshape(NBLK, RB, EIG_IN)
                 .transpose(0, 2, 1).reshape(1, B * EIG_IN))
    w_big = jnp.tile(w[:, None], (1, 128))

    y_tab, vals, idx = _tc_stage(lhsT_bf, w1_bf, ev2_bf, in_w_perm, w_big)
    y_tab = y_tab.reshape(B * EIG_IN * NUM_EIG // 2, 2 * DIM_Y)
    rows = jax.lax.shift_right_logical(idx, 1)
    yg128 = _sc_gather(y_tab, rows)
    out = _asm_stage(yg128, idx, vals)
    return out.reshape(B, DIM_Y + 1, DIM_Y)
